# baseline (speedup 1.0000x reference)
import functools

import jax
import jax.numpy as jnp
from jax import lax
from jax.experimental import pallas as pl
from jax.experimental.pallas import tpu as pltpu
from jax.experimental.pallas import tpu_sc as plsc

B, A, F = 256, 128, 128
NW = 32
GPW = B // NW
NV = F // 16
ZR = 64
IN_BITS = (128, 64, 32, 16, 8)
Z_BITS = (64, 32, 16, 8)


def _sc_body(x_hbm, ms_hbm, prev_hbm, out_hbm,
             ms_v, xb, p0b, p1b, zb, ms_s, sem_x, sem_p, sem_out, sem_z):
    wid = lax.axis_index("s") * 2 + lax.axis_index("c")
    g0 = pl.multiple_of(wid * GPW, GPW)
    pltpu.sync_copy(ms_hbm.at[pl.ds(g0 * 2, 2 * GPW)], ms_v)
    mvec = ms_v[...]
    for i in range(GPW):
        ms_s[i] = mvec[2 * i]

    zvec = jnp.zeros((16,), jnp.float32)

    def zinit(j, _):
        for k in range(NV):
            zb[j, pl.ds(k * 16, 16)] = zvec
        return 0

    lax.fori_loop(0, ZR, zinit, 0)

    def r8_of(idx):
        return (ms_s[idx] + 7) & ~7

    def in_po2(idx, op):
        g = g0 + idx
        r8 = r8_of(idx)
        sx = sem_x.at[lax.rem(idx, 3)]
        sp = sem_p.at[lax.rem(idx, 2)]
        bx = xb.at[lax.rem(idx, 3)]
        b0 = p0b.at[lax.rem(idx, 2)]
        b1 = p1b.at[lax.rem(idx, 2)]
        for bit in IN_BITS:
            def blk(bit=bit):
                off = pl.multiple_of(r8 & ~(2 * bit - 1), 8)
                sl = pl.ds(off, bit)
                op(pltpu.make_async_copy(x_hbm.at[g, sl, :], bx.at[sl], sx))
                op(pltpu.make_async_copy(prev_hbm.at[0, g, sl, :],
                                         b0.at[sl], sp))
                op(pltpu.make_async_copy(prev_hbm.at[1, g, sl, :],
                                         b1.at[sl], sp))
            pl.when((r8 & bit) != 0)(blk)

    def out_po2(idx, op):
        g = g0 + idx
        r8 = r8_of(idx)
        s = sem_out.at[lax.rem(idx, 3)]
        bx = xb.at[lax.rem(idx, 3)]
        for bit in IN_BITS:
            def blk(bit=bit):
                off = pl.multiple_of(r8 & ~(2 * bit - 1), 8)
                sl = pl.ds(off, bit)
                op(pltpu.make_async_copy(bx.at[sl], out_hbm.at[g, sl, :], s))
            pl.when((r8 & bit) != 0)(blk)

    def z_po2(idx, op):
        g = g0 + idx
        r8 = r8_of(idx)
        t = A - r8
        for bit in Z_BITS:
            def blk(bit=bit):
                off = pl.multiple_of(r8 + (t & ~(2 * bit - 1)), 8)
                op(pltpu.make_async_copy(
                    zb.at[pl.ds(0, bit), :],
                    out_hbm.at[g, pl.ds(off, bit), :], sem_z))
            pl.when((t & bit) != 0)(blk)

    def compute(idx):
        m = ms_s[idx]
        ngrp = (m + 7) >> 3
        s = lax.rem(idx, 3)
        p = lax.rem(idx, 2)

        def grp_body(grp, _):
            j8 = grp * 8
            for jj in range(8):
                j = j8 + jj
                valid = j < m
                for k in range(NV):
                    sl = pl.ds(k * 16, 16)
                    v = xb[s, j, sl] + p0b[p, j, sl] + p1b[p, j, sl]
                    xb[s, j, sl] = jnp.where(valid, jnp.maximum(v, 0.0), 0.0)
            return 0

        lax.fori_loop(0, ngrp, grp_body, 0)

    start = lambda cp: cp.start()
    wait = lambda cp: cp.wait()

    in_po2(0, start)

    def graph_body(i, _):
        pl.when(i >= 2)(lambda: out_po2(i - 2, wait))
        pl.when(i >= 1)(lambda: z_po2(i - 1, wait))
        pl.when(i + 1 < GPW)(lambda: in_po2(i + 1, start))
        in_po2(i, wait)
        compute(i)
        out_po2(i, start)
        z_po2(i, start)
        return 0

    lax.fori_loop(0, GPW, graph_body, 0)

    out_po2(GPW - 2, wait)
    out_po2(GPW - 1, wait)
    z_po2(GPW - 1, wait)


def kernel(atom_features, mol_slice, prev_activations):
    mesh = plsc.VectorSubcoreMesh(core_axis_name="c", subcore_axis_name="s")
    run = functools.partial(
        pl.kernel,
        mesh=mesh,
        out_type=jax.ShapeDtypeStruct((B, A, F), jnp.float32),
        scratch_types=[
            pltpu.VMEM((2 * GPW,), jnp.int32),
            pltpu.VMEM((3, A, F), jnp.float32),
            pltpu.VMEM((2, A, F), jnp.float32),
            pltpu.VMEM((2, A, F), jnp.float32),
            pltpu.VMEM((ZR, F), jnp.float32),
            pltpu.SMEM((GPW,), jnp.int32),
            pltpu.SemaphoreType.DMA((3,)),
            pltpu.SemaphoreType.DMA((2,)),
            pltpu.SemaphoreType.DMA((3,)),
            pltpu.SemaphoreType.DMA,
        ],
    )(_sc_body)
    return run(atom_features, mol_slice.reshape(-1), prev_activations)


# speedup vs baseline: 1.7843x; 1.7843x over previous
import functools

import jax
import jax.numpy as jnp
from jax import lax
from jax.experimental import pallas as pl
from jax.experimental.pallas import tpu as pltpu
from jax.experimental.pallas import tpu_sc as plsc

B, A, F = 256, 128, 128
NW = 32
GPW = B // NW
NV = F // 16
ZR = 64
IN_BITS = (128, 64, 32, 16, 8)
Z_BITS = (64, 32, 16, 8)


def _sc_body(x_hbm, ms_hbm, prev_hbm, out_hbm,
             ms_v, xb, p0b, p1b, ob, zb, ms_s, sem_in, sem_out, sem_z):
    wid = lax.axis_index("s") * 2 + lax.axis_index("c")
    g0 = pl.multiple_of(wid * GPW, GPW)
    pltpu.sync_copy(ms_hbm.at[pl.ds(g0 * 2, 2 * GPW)], ms_v)
    mvec = ms_v[...]
    for i in range(GPW):
        ms_s[i] = mvec[2 * i]

    zvec = jnp.zeros((16,), jnp.float32)

    def zinit(j, _):
        for k in range(NV):
            zb[j, pl.ds(k * 16, 16)] = zvec
        return 0

    lax.fori_loop(0, ZR, zinit, 0)

    def r8_of(idx):
        return (ms_s[idx] + 7) & ~7

    def in_po2(idx, op):
        g = g0 + idx
        r8 = r8_of(idx)
        s = lax.rem(idx, 2)
        sem = sem_in.at[s]
        for bit in IN_BITS:
            def blk(bit=bit):
                off = pl.multiple_of(r8 & ~(2 * bit - 1), 8)
                sl = pl.ds(off, bit)
                op(pltpu.make_async_copy(x_hbm.at[g, sl, :],
                                         xb.at[s, sl], sem))
                op(pltpu.make_async_copy(prev_hbm.at[0, g, sl, :],
                                         p0b.at[s, sl], sem))
                op(pltpu.make_async_copy(prev_hbm.at[1, g, sl, :],
                                         p1b.at[s, sl], sem))
            pl.when((r8 & bit) != 0)(blk)

    def out_po2(idx, op):
        g = g0 + idx
        r8 = r8_of(idx)
        for bit in IN_BITS:
            def blk(bit=bit):
                off = pl.multiple_of(r8 & ~(2 * bit - 1), 8)
                sl = pl.ds(off, bit)
                op(pltpu.make_async_copy(ob.at[sl], out_hbm.at[g, sl, :],
                                         sem_out))
            pl.when((r8 & bit) != 0)(blk)

    def z_po2(idx, op):
        g = g0 + idx
        r8 = r8_of(idx)
        t = A - r8
        for bit in Z_BITS:
            def blk(bit=bit):
                off = pl.multiple_of(r8 + (t & ~(2 * bit - 1)), 8)
                op(pltpu.make_async_copy(
                    zb.at[pl.ds(0, bit), :],
                    out_hbm.at[g, pl.ds(off, bit), :], sem_z))
            pl.when((t & bit) != 0)(blk)

    def compute(idx):
        m = ms_s[idx]
        r8 = r8_of(idx)
        s = lax.rem(idx, 2)

        @plsc.parallel_loop(0, r8, step=1, unroll=8)
        def row_body(j):
            valid = j < m
            for k in range(NV):
                sl = pl.ds(k * 16, 16)
                v = xb[s, j, sl] + p0b[s, j, sl] + p1b[s, j, sl]
                ob[j, sl] = jnp.where(valid, jnp.maximum(v, 0.0), 0.0)

    start = lambda cp: cp.start()
    wait = lambda cp: cp.wait()

    in_po2(0, start)

    def graph_body(i, _):
        pl.when(i >= 1)(lambda: z_po2(i - 1, wait))
        pl.when(i + 1 < GPW)(lambda: in_po2(i + 1, start))
        in_po2(i, wait)
        pl.when(i >= 1)(lambda: out_po2(i - 1, wait))
        compute(i)
        out_po2(i, start)
        z_po2(i, start)
        return 0

    lax.fori_loop(0, GPW, graph_body, 0)

    out_po2(GPW - 1, wait)
    z_po2(GPW - 1, wait)


def kernel(atom_features, mol_slice, prev_activations):
    mesh = plsc.VectorSubcoreMesh(core_axis_name="c", subcore_axis_name="s")
    run = functools.partial(
        pl.kernel,
        mesh=mesh,
        out_type=jax.ShapeDtypeStruct((B, A, F), jnp.float32),
        scratch_types=[
            pltpu.VMEM((2 * GPW,), jnp.int32),
            pltpu.VMEM((2, A, F), jnp.float32),
            pltpu.VMEM((2, A, F), jnp.float32),
            pltpu.VMEM((2, A, F), jnp.float32),
            pltpu.VMEM((A, F), jnp.float32),
            pltpu.VMEM((ZR, F), jnp.float32),
            pltpu.SMEM((GPW,), jnp.int32),
            pltpu.SemaphoreType.DMA((2,)),
            pltpu.SemaphoreType.DMA,
            pltpu.SemaphoreType.DMA,
        ],
    )(_sc_body)
    return run(atom_features, mol_slice.reshape(-1), prev_activations)


# speedup vs baseline: 1.7960x; 1.0066x over previous
import functools

import jax
import jax.numpy as jnp
from jax import lax
from jax.experimental import pallas as pl
from jax.experimental.pallas import tpu as pltpu
from jax.experimental.pallas import tpu_sc as plsc

B, A, F = 256, 128, 128
NW = 32
GPW = B // NW
NV = F // 16
ZR = 64
IN_BITS = (128, 64, 32, 16, 8)
Z_BITS = (64, 32, 16, 8)


def _sc_body(x_hbm, ms_hbm, prev_hbm, out_hbm,
             ms_v, xb, p0b, p1b, ob, ms_s, sem_in, sem_out):
    wid = lax.axis_index("s") * 2 + lax.axis_index("c")
    g0 = pl.multiple_of(wid * GPW, GPW)
    pltpu.sync_copy(ms_hbm.at[pl.ds(g0 * 2, 2 * GPW)], ms_v)
    mvec = ms_v[...]
    for i in range(GPW):
        ms_s[i] = mvec[2 * i]

    def r8_of(idx):
        return (ms_s[idx] + 7) & ~7

    def in_po2(idx, op):
        g = g0 + idx
        r8 = r8_of(idx)
        s = lax.rem(idx, 2)
        sem = sem_in.at[s]
        for bit in IN_BITS:
            def blk(bit=bit):
                off = pl.multiple_of(r8 & ~(2 * bit - 1), 8)
                sl = pl.ds(off, bit)
                op(pltpu.make_async_copy(x_hbm.at[g, sl, :],
                                         xb.at[s, sl], sem))
                op(pltpu.make_async_copy(prev_hbm.at[0, g, sl, :],
                                         p0b.at[s, sl], sem))
                op(pltpu.make_async_copy(prev_hbm.at[1, g, sl, :],
                                         p1b.at[s, sl], sem))
            pl.when((r8 & bit) != 0)(blk)

    def out_full(idx, op):
        op(pltpu.make_async_copy(ob, out_hbm.at[g0 + idx], sem_out))

    def compute(idx):
        m = ms_s[idx]
        r8 = r8_of(idx)
        prev_end = jnp.where(idx == 0, A, r8_of(lax.rem(idx - 1 + GPW, GPW)))
        zend = jnp.maximum(r8, prev_end)
        zvec = jnp.zeros((16,), jnp.float32)

        @plsc.parallel_loop(r8, zend, step=1, unroll=4)
        def zrow_body(j):
            for k in range(NV):
                ob[j, pl.ds(k * 16, 16)] = zvec

        s = lax.rem(idx, 2)

        @plsc.parallel_loop(0, r8, step=1, unroll=8)
        def row_body(j):
            valid = j < m
            for k in range(NV):
                sl = pl.ds(k * 16, 16)
                v = xb[s, j, sl] + p0b[s, j, sl] + p1b[s, j, sl]
                ob[j, sl] = jnp.where(valid, jnp.maximum(v, 0.0), 0.0)

    start = lambda cp: cp.start()
    wait = lambda cp: cp.wait()

    in_po2(0, start)

    def graph_body(i, _):
        pl.when(i + 1 < GPW)(lambda: in_po2(i + 1, start))
        in_po2(i, wait)
        pl.when(i >= 1)(lambda: out_full(i - 1, wait))
        compute(i)
        out_full(i, start)
        return 0

    lax.fori_loop(0, GPW, graph_body, 0)

    out_full(GPW - 1, wait)


def kernel(atom_features, mol_slice, prev_activations):
    mesh = plsc.VectorSubcoreMesh(core_axis_name="c", subcore_axis_name="s")
    run = functools.partial(
        pl.kernel,
        mesh=mesh,
        out_type=jax.ShapeDtypeStruct((B, A, F), jnp.float32),
        scratch_types=[
            pltpu.VMEM((2 * GPW,), jnp.int32),
            pltpu.VMEM((2, A, F), jnp.float32),
            pltpu.VMEM((2, A, F), jnp.float32),
            pltpu.VMEM((2, A, F), jnp.float32),
            pltpu.VMEM((A, F), jnp.float32),
            pltpu.SMEM((GPW,), jnp.int32),
            pltpu.SemaphoreType.DMA((2,)),
            pltpu.SemaphoreType.DMA,
        ],
    )(_sc_body)
    return run(atom_features, mol_slice.reshape(-1), prev_activations)
